# baseline (speedup 1.0000x reference)
import functools

import jax
import jax.numpy as jnp
from jax import lax
from jax.experimental import pallas as pl
from jax.experimental.pallas import tpu as pltpu
from jax.experimental.pallas import tpu_sc as plsc

_N = 4096
_K = 30
_H = 3
_NEG = 0.2
_NW = 32
_NPW = _N // _NW


def _knn_body(xb_ref, xt_ref, brow_ref, bcol_ref, o_ref):
    i = pl.program_id(0)
    blk = xb_ref.shape[0]
    xb = xb_ref[...]
    xt = xt_ref[...]
    sqrow = jnp.sum(xb * xb, axis=1, keepdims=True)
    sqcol = jnp.sum(xt * xt, axis=0, keepdims=True)
    d2 = sqrow + sqcol - 2.0 * jnp.dot(xb, xt, preferred_element_type=jnp.float32)
    col = jax.lax.broadcasted_iota(jnp.int32, (blk, _N), 1)
    row = jax.lax.broadcasted_iota(jnp.int32, (blk, _N), 0) + i * blk
    cross = brow_ref[...] != bcol_ref[...]
    d2 = jnp.where(cross | (col == row), jnp.inf, d2)
    cols = []
    big = jnp.int32(2 ** 30)
    for _ in range(_K):
        m = jnp.min(d2, axis=1, keepdims=True)
        cand = jnp.where(d2 == m, col, big)
        c = jnp.min(cand, axis=1, keepdims=True)
        cols.append(c)
        d2 = jnp.where(cand == c, jnp.inf, d2)
    o_ref[...] = jnp.concatenate(cols, axis=1)


def _knn_idx(xf, batch):
    d = xf.shape[1]
    dp = 8 if d < 8 else d
    if d != dp:
        xf = jnp.pad(xf, ((0, 0), (0, dp - d)))
    blk = 512
    return pl.pallas_call(
        _knn_body,
        grid=(_N // blk,),
        in_specs=[
            pl.BlockSpec((blk, dp), lambda i: (i, 0)),
            pl.BlockSpec((dp, _N), lambda i: (0, 0)),
            pl.BlockSpec((blk, 1), lambda i: (i, 0)),
            pl.BlockSpec((1, _N), lambda i: (0, 0)),
        ],
        out_specs=pl.BlockSpec((blk, _K), lambda i: (i, 0)),
        out_shape=jax.ShapeDtypeStruct((_N, _K), jnp.int32),
    )(xf, xf.T, batch.reshape(_N, 1), batch.reshape(1, _N))


def _gat_pre_body(link_ref, w_ref, asv_ref, adv_ref, h_ref, asrc_ref, adst_ref):
    hh = jnp.dot(link_ref[...], w_ref[...], preferred_element_type=jnp.float32)
    h_ref[...] = hh
    c = hh.shape[1] // _H
    srcs, dsts = [], []
    for h in range(_H):
        blkc = hh[:, h * c:(h + 1) * c]
        srcs.append(jnp.sum(blkc * asv_ref[:, h * c:(h + 1) * c], axis=1, keepdims=True))
        dsts.append(jnp.sum(blkc * adv_ref[:, h * c:(h + 1) * c], axis=1, keepdims=True))
    asrc_ref[...] = jnp.concatenate(srcs, axis=1)
    adst_ref[...] = jnp.concatenate(dsts, axis=1)


def _gat_agg_body(idx_ref, asrcT_ref, adst_ref, h_ref, b_ref, wm_ref, bm_ref,
                  o_ref):
    i = pl.program_id(0)
    blk = idx_ref.shape[0]
    col = jax.lax.broadcasted_iota(jnp.int32, (blk, _N), 1)
    row = jax.lax.broadcasted_iota(jnp.int32, (blk, _N), 0) + i * blk
    mask = col == row
    for j in range(_K):
        mask = mask | (col == idx_ref[:, j:j + 1])
    c = h_ref.shape[1] // _H
    gs = []
    for h in range(_H):
        af = asrcT_ref[h:h + 1, :] + adst_ref[:, h:h + 1]
        af = jnp.where(af >= 0.0, af, _NEG * af)
        amax = jnp.max(jnp.where(mask, af, -jnp.inf), axis=1, keepdims=True)
        e = jnp.where(mask, jnp.exp(af - amax), 0.0)
        p = e / (jnp.sum(e, axis=1, keepdims=True) + 1e-16)
        gs.append(jnp.dot(p, h_ref[:, h * c:(h + 1) * c],
                          preferred_element_type=jnp.float32))
    g = jnp.concatenate(gs, axis=1) + b_ref[...]
    o_ref[...] = jnp.dot(g, wm_ref[...], preferred_element_type=jnp.float32) + bm_ref[...]


def _sc_agg_body_factory(hc, widths, ring):
    nv = hc // 16
    cph = hc // _H // 16
    nt = len(widths)

    def body(*refs):
        idx_hbm, a0_hbm, a1_hbm, a2_hbm, d0_hbm, d1_hbm, d2_hbm = refs[:7]
        h_hbms = refs[7:7 + nt]
        g_hbm = refs[7 + nt]
        sc = refs[8 + nt:]
        idx_v, a0_v, a1_v, a2_v, d0_v, d1_v, d2_v, wtmp_v, out_v = sc[:9]
        rbs = sc[9:9 + ring * nt]
        sems = sc[9 + ring * nt:]
        wid = lax.axis_index("s") * 2 + lax.axis_index("c")
        base = wid * _NPW
        pltpu.sync_copy(idx_hbm.at[pl.ds(base, _NPW)], idx_v)
        pltpu.sync_copy(a0_hbm, a0_v)
        pltpu.sync_copy(a1_hbm, a1_v)
        pltpu.sync_copy(a2_hbm, a2_v)
        pltpu.sync_copy(d0_hbm.at[pl.ds(base, _NPW)], d0_v)
        pltpu.sync_copy(d1_hbm.at[pl.ds(base, _NPW)], d1_v)
        pltpu.sync_copy(d2_hbm.at[pl.ds(base, _NPW)], d2_v)
        for slot in range(ring):
            for t in range(nt):
                pltpu.make_async_copy(h_hbms[t].at[idx_v.at[slot]],
                                      rbs[slot * nt + t],
                                      sems[slot * nt + t]).start()
        li = lax.iota(jnp.int32, 16)
        srcs = [a0_v, a1_v, a2_v]
        dsts = [d0_v, d1_v, d2_v]

        def node_step(n, slot):
            i0 = idx_v[n, pl.ds(0, 16)]
            i1 = idx_v[n, pl.ds(16, 16)]
            nfull = jnp.full((16,), 0, jnp.int32) + n
            for h in range(_H):
                g0 = plsc.load_gather(srcs[h], [i0])
                g1 = plsc.load_gather(srcs[h], [i1])
                dsp = plsc.load_gather(dsts[h], [nfull])
                al0 = g0 + dsp
                al0 = jnp.where(al0 >= 0.0, al0, _NEG * al0)
                al1 = g1 + dsp
                al1 = jnp.where(al1 >= 0.0, al1, _NEG * al1)
                al1 = jnp.where(li == 15, -jnp.inf, al1)
                m = jnp.maximum(jnp.max(al0), jnp.max(al1))
                e0 = jnp.exp(al0 - m)
                e1 = jnp.where(li == 15, 0.0, jnp.exp(al1 - m))
                s = jnp.sum(e0) + jnp.sum(e1) + 1e-16
                wtmp_v[pl.ds(h * 32, 16)] = e0 / s
                wtmp_v[pl.ds(h * 32 + 16, 16)] = e1 / s
            for t in range(nt):
                pltpu.make_async_copy(h_hbms[t].at[idx_v.at[n]],
                                      rbs[slot * nt + t],
                                      sems[slot * nt + t]).wait()
            accs = [jnp.zeros((16,), jnp.float32) for _ in range(nv)]
            for j in range(32):
                ws = [plsc.load_gather(wtmp_v, [jnp.full((16,), h * 32 + j, jnp.int32)])
                      for h in range(_H)]
                c0 = 0
                for t in range(nt):
                    wt = widths[t] // 16
                    for cl in range(min(wt, nv - c0)):
                        c = c0 + cl
                        accs[c] = accs[c] + ws[c // cph] * rbs[slot * nt + t][j, pl.ds(cl * 16, 16)]
                    c0 += wt
            for c in range(nv):
                out_v[n, pl.ds(c * 16, 16)] = accs[c]

            @pl.when(n + ring < _NPW)
            def _():
                for t in range(nt):
                    pltpu.make_async_copy(h_hbms[t].at[idx_v.at[n + ring]],
                                          rbs[slot * nt + t],
                                          sems[slot * nt + t]).start()

        def outer(k, carry):
            for slot in range(ring):
                node_step(ring * k + slot, slot)
            return carry

        lax.fori_loop(0, _NPW // ring, outer, 0)
        pltpu.sync_copy(out_v, g_hbm.at[pl.ds(base, _NPW)])

    return body


def _sc_agg(idx32, asrc, adst, h):
    hc = h.shape[1]
    hp = ((hc + 127) // 128) * 128
    widths, ring = (hp,), 2
    hs = (jnp.pad(h, ((0, 0), (0, hp - hc))) if hp != hc else h,)
    nt = len(widths)
    mesh = plsc.VectorSubcoreMesh(core_axis_name="c", subcore_axis_name="s")
    k = functools.partial(
        pl.kernel,
        mesh=mesh,
        compiler_params=pltpu.CompilerParams(needs_layout_passes=False),
        out_type=jax.ShapeDtypeStruct((_N, hc), jnp.float32),
        scratch_types=[
            pltpu.VMEM((_NPW, 32), jnp.int32),
            pltpu.VMEM((_N,), jnp.float32),
            pltpu.VMEM((_N,), jnp.float32),
            pltpu.VMEM((_N,), jnp.float32),
            pltpu.VMEM((_NPW,), jnp.float32),
            pltpu.VMEM((_NPW,), jnp.float32),
            pltpu.VMEM((_NPW,), jnp.float32),
            pltpu.VMEM((96,), jnp.float32),
            pltpu.VMEM((_NPW, hc), jnp.float32),
        ] + [pltpu.VMEM((32, widths[t]), jnp.float32)
             for _ in range(ring) for t in range(nt)]
          + [pltpu.SemaphoreType.DMA] * (ring * nt),
    )(_sc_agg_body_factory(hc, widths, ring))
    a0, a1, a2 = (asrc[:, i] for i in range(_H))
    d0, d1, d2 = (adst[:, i] for i in range(_H))
    return k(idx32, a0, a1, a2, d0, d1, d2, *hs)


def _proj_body(g_ref, b_ref, wm_ref, bm_ref, o_ref):
    o_ref[...] = jnp.dot(g_ref[...] + b_ref[...], wm_ref[...],
                         preferred_element_type=jnp.float32) + bm_ref[...]


def _gat_sc(xf, idx32, W, att_src, att_dst, bias, out_ch, Wm, bm):
    cin = xf.shape[1]
    hc = _H * out_ch
    blk = 256
    h, asrc, adst = pl.pallas_call(
        _gat_pre_body,
        grid=(_N // blk,),
        in_specs=[
            pl.BlockSpec((blk, cin), lambda i: (i, 0)),
            pl.BlockSpec((cin, hc), lambda i: (0, 0)),
            pl.BlockSpec((1, hc), lambda i: (0, 0)),
            pl.BlockSpec((1, hc), lambda i: (0, 0)),
        ],
        out_specs=[
            pl.BlockSpec((blk, hc), lambda i: (i, 0)),
            pl.BlockSpec((blk, _H), lambda i: (i, 0)),
            pl.BlockSpec((blk, _H), lambda i: (i, 0)),
        ],
        out_shape=[
            jax.ShapeDtypeStruct((_N, hc), jnp.float32),
            jax.ShapeDtypeStruct((_N, _H), jnp.float32),
            jax.ShapeDtypeStruct((_N, _H), jnp.float32),
        ],
    )(xf, W, att_src.reshape(1, hc), att_dst.reshape(1, hc))

    g = _sc_agg(idx32, asrc, adst, h)

    out_ch2 = Wm.shape[1]
    return pl.pallas_call(
        _proj_body,
        grid=(_N // blk,),
        in_specs=[
            pl.BlockSpec((blk, hc), lambda i: (i, 0)),
            pl.BlockSpec((1, hc), lambda i: (0, 0)),
            pl.BlockSpec((hc, out_ch2), lambda i: (0, 0)),
            pl.BlockSpec((1, out_ch2), lambda i: (0, 0)),
        ],
        out_specs=pl.BlockSpec((blk, out_ch2), lambda i: (i, 0)),
        out_shape=jax.ShapeDtypeStruct((_N, out_ch2), jnp.float32),
    )(g, bias.reshape(1, hc), Wm, bm.reshape(1, out_ch2))


def _gat(xf, idx, W, att_src, att_dst, bias, out_ch, Wm, bm):
    cin = xf.shape[1]
    hc = _H * out_ch
    blk = 256
    h, asrc, adst = pl.pallas_call(
        _gat_pre_body,
        grid=(_N // blk,),
        in_specs=[
            pl.BlockSpec((blk, cin), lambda i: (i, 0)),
            pl.BlockSpec((cin, hc), lambda i: (0, 0)),
            pl.BlockSpec((1, hc), lambda i: (0, 0)),
            pl.BlockSpec((1, hc), lambda i: (0, 0)),
        ],
        out_specs=[
            pl.BlockSpec((blk, hc), lambda i: (i, 0)),
            pl.BlockSpec((blk, _H), lambda i: (i, 0)),
            pl.BlockSpec((blk, _H), lambda i: (i, 0)),
        ],
        out_shape=[
            jax.ShapeDtypeStruct((_N, hc), jnp.float32),
            jax.ShapeDtypeStruct((_N, _H), jnp.float32),
            jax.ShapeDtypeStruct((_N, _H), jnp.float32),
        ],
    )(xf, W, att_src.reshape(1, hc), att_dst.reshape(1, hc))

    out_ch2 = Wm.shape[1]
    return pl.pallas_call(
        _gat_agg_body,
        grid=(_N // blk,),
        in_specs=[
            pl.BlockSpec((blk, _K), lambda i: (i, 0)),
            pl.BlockSpec((_H, _N), lambda i: (0, 0)),
            pl.BlockSpec((blk, _H), lambda i: (i, 0)),
            pl.BlockSpec((_N, hc), lambda i: (0, 0)),
            pl.BlockSpec((1, hc), lambda i: (0, 0)),
            pl.BlockSpec((hc, out_ch2), lambda i: (0, 0)),
            pl.BlockSpec((1, out_ch2), lambda i: (0, 0)),
        ],
        out_specs=pl.BlockSpec((blk, out_ch2), lambda i: (i, 0)),
        out_shape=jax.ShapeDtypeStruct((_N, out_ch2), jnp.float32),
    )(idx, asrc.T, adst, h, bias.reshape(1, hc), Wm, bm.reshape(1, out_ch2))


def _head_a_body(link_ref, f1_ref, fb1_ref, f2_ref, fb2_ref, o_ref):
    i = pl.program_id(0)
    t = jnp.dot(link_ref[...], f1_ref[...], preferred_element_type=jnp.float32)
    t = jnp.maximum(t + fb1_ref[...], 0.0)
    x5 = jnp.dot(t, f2_ref[...], preferred_element_type=jnp.float32) + fb2_ref[...]
    bmax = jnp.max(x5, axis=0, keepdims=True)

    @pl.when(i == 0)
    def _():
        o_ref[...] = bmax

    @pl.when(i > 0)
    def _():
        o_ref[...] = jnp.maximum(o_ref[...], bmax)


def _head_b_body(link_ref, g_ref, m1a_ref, m1b_ref, mb1_ref, m2_ref, mb2_ref,
                 m3_ref, mb3_ref, m4_ref, mb4_ref, o_ref):
    g2 = jnp.dot(g_ref[...], m1b_ref[...], preferred_element_type=jnp.float32)
    h = jnp.dot(link_ref[...], m1a_ref[...], preferred_element_type=jnp.float32)
    h = jnp.maximum(h + g2 + mb1_ref[...], 0.0)
    h = jnp.dot(h, m2_ref[...], preferred_element_type=jnp.float32)
    h = jnp.maximum(h + mb2_ref[...], 0.0)
    h = jnp.dot(h, m3_ref[...], preferred_element_type=jnp.float32)
    h = jnp.maximum(h + mb3_ref[...], 0.0)
    o = jnp.dot(h, m4_ref[...], preferred_element_type=jnp.float32) + mb4_ref[...]
    m = jnp.max(o, axis=1, keepdims=True)
    lse = jnp.log(jnp.sum(jnp.exp(o - m), axis=1, keepdims=True))
    o_ref[...] = o - m - lse


def _mlp_head(link4, F1, fb1, F2, fb2, M1, mb1, M2, mb2, M3, mb3, M4, mb4):
    blk = 256
    nblk = _N // blk
    cin = link4.shape[1]
    gfeat = pl.pallas_call(
        _head_a_body,
        grid=(nblk,),
        in_specs=[
            pl.BlockSpec((blk, cin), lambda i: (i, 0)),
            pl.BlockSpec((cin, 1024), lambda i: (0, 0)),
            pl.BlockSpec((1, 1024), lambda i: (0, 0)),
            pl.BlockSpec((1024, 1024), lambda i: (0, 0)),
            pl.BlockSpec((1, 1024), lambda i: (0, 0)),
        ],
        out_specs=pl.BlockSpec((1, 1024), lambda i: (0, 0)),
        out_shape=jax.ShapeDtypeStruct((1, 1024), jnp.float32),
    )(link4, F1, fb1.reshape(1, -1), F2, fb2.reshape(1, -1))

    M1a, M1b = M1[:cin], M1[cin:]
    out = pl.pallas_call(
        _head_b_body,
        grid=(nblk,),
        in_specs=[
            pl.BlockSpec((blk, cin), lambda i: (i, 0)),
            pl.BlockSpec((1, 1024), lambda i: (0, 0)),
            pl.BlockSpec((cin, 256), lambda i: (0, 0)),
            pl.BlockSpec((1024, 256), lambda i: (0, 0)),
            pl.BlockSpec((1, 256), lambda i: (0, 0)),
            pl.BlockSpec((256, 256), lambda i: (0, 0)),
            pl.BlockSpec((1, 256), lambda i: (0, 0)),
            pl.BlockSpec((256, 128), lambda i: (0, 0)),
            pl.BlockSpec((1, 128), lambda i: (0, 0)),
            pl.BlockSpec((128, 50), lambda i: (0, 0)),
            pl.BlockSpec((1, 50), lambda i: (0, 0)),
        ],
        out_specs=pl.BlockSpec((blk, 50), lambda i: (i, 0)),
        out_shape=jax.ShapeDtypeStruct((_N, 50), jnp.float32),
    )(link4, gfeat, M1a, M1b, mb1.reshape(1, -1), M2, mb2.reshape(1, -1),
      M3, mb3.reshape(1, -1), M4, mb4.reshape(1, -1))
    return out


def kernel(x, pos, batch, W1, as1, ad1, b1, Wm1, bm1, W2, as2, ad2, b2, Wm2,
           bm2, W3, as3, ad3, b3, Wm3, bm3, W4, as4, ad4, b4, Wm4, bm4, F1,
           fb1, F2, fb2, M1, mb1, M2, mb2, M3, mb3, M4, mb4):
    ar = jnp.arange(_N, dtype=jnp.int32).reshape(_N, 1)
    x0 = jnp.concatenate([x, pos], axis=-1)
    idx32 = jnp.concatenate([_knn_idx(x0, batch), ar, ar], axis=1)
    x1 = _gat_sc(x0, idx32, W1, as1, ad1, b1, 64, Wm1, bm1)
    idx32 = jnp.concatenate([_knn_idx(x1, batch), ar, ar], axis=1)
    link1 = jnp.concatenate([x0, x1], axis=1)
    x2 = _gat_sc(link1, idx32, W2, as2, ad2, b2, 64, Wm2, bm2)
    idx32 = jnp.concatenate([_knn_idx(x2, batch), ar, ar], axis=1)
    link2 = jnp.concatenate([x0, x1, x2], axis=1)
    x3 = _gat_sc(link2, idx32, W3, as3, ad3, b3, 64, Wm3, bm3)
    link3 = jnp.concatenate([x0, x1, x2, x3], axis=1)
    x4 = _gat_sc(link3, idx32, W4, as4, ad4, b4, 128, Wm4, bm4)
    link4 = jnp.concatenate([x0, x1, x2, x3, x4], axis=-1)
    return _mlp_head(link4, F1, fb1, F2, fb2, M1, mb1, M2, mb2, M3, mb3, M4, mb4)


# speedup vs baseline: 1.0555x; 1.0555x over previous
import functools

import jax
import jax.numpy as jnp
from jax import lax
from jax.experimental import pallas as pl
from jax.experimental.pallas import tpu as pltpu
from jax.experimental.pallas import tpu_sc as plsc

_N = 4096
_K = 30
_H = 3
_NEG = 0.2
_NW = 32
_NPW = _N // _NW


def _knn_body(xb_ref, xt_ref, brow_ref, bcol_ref, o_ref):
    i = pl.program_id(0)
    blk = xb_ref.shape[0]
    xb = xb_ref[...]
    xt = xt_ref[...]
    sqrow = jnp.sum(xb * xb, axis=1, keepdims=True)
    sqcol = jnp.sum(xt * xt, axis=0, keepdims=True)
    d2 = sqrow + sqcol - 2.0 * jnp.dot(xb, xt, preferred_element_type=jnp.float32)
    col = jax.lax.broadcasted_iota(jnp.int32, (blk, _N), 1)
    row = jax.lax.broadcasted_iota(jnp.int32, (blk, _N), 0) + i * blk
    cross = brow_ref[...] != bcol_ref[...]
    d2 = jnp.where(cross | (col == row), jnp.inf, d2)
    cols = []
    big = jnp.int32(2 ** 30)
    for _ in range(_K):
        m = jnp.min(d2, axis=1, keepdims=True)
        cand = jnp.where(d2 == m, col, big)
        c = jnp.min(cand, axis=1, keepdims=True)
        cols.append(c)
        d2 = jnp.where(cand == c, jnp.inf, d2)
    o_ref[...] = jnp.concatenate(cols, axis=1)


def _knn_idx(xf, batch):
    d = xf.shape[1]
    dp = 8 if d < 8 else d
    if d != dp:
        xf = jnp.pad(xf, ((0, 0), (0, dp - d)))
    blk = 256
    return pl.pallas_call(
        _knn_body,
        grid=(_N // blk,),
        in_specs=[
            pl.BlockSpec((blk, dp), lambda i: (i, 0)),
            pl.BlockSpec((dp, _N), lambda i: (0, 0)),
            pl.BlockSpec((blk, 1), lambda i: (i, 0)),
            pl.BlockSpec((1, _N), lambda i: (0, 0)),
        ],
        out_specs=pl.BlockSpec((blk, _K), lambda i: (i, 0)),
        out_shape=jax.ShapeDtypeStruct((_N, _K), jnp.int32),
    )(xf, xf.T, batch.reshape(_N, 1), batch.reshape(1, _N))


def _gat_pre_body(link_ref, w_ref, asv_ref, adv_ref, h_ref, asrc_ref, adst_ref):
    hh = jnp.dot(link_ref[...], w_ref[...], preferred_element_type=jnp.float32)
    h_ref[...] = hh
    c = hh.shape[1] // _H
    srcs, dsts = [], []
    for h in range(_H):
        blkc = hh[:, h * c:(h + 1) * c]
        srcs.append(jnp.sum(blkc * asv_ref[:, h * c:(h + 1) * c], axis=1, keepdims=True))
        dsts.append(jnp.sum(blkc * adv_ref[:, h * c:(h + 1) * c], axis=1, keepdims=True))
    asrc_ref[...] = jnp.concatenate(srcs, axis=1)
    adst_ref[...] = jnp.concatenate(dsts, axis=1)


def _sc_agg_body_factory(hc, widths, ring):
    nv = hc // 16
    cph = hc // _H // 16
    nt = len(widths)

    def body(*refs):
        idx_hbm, a0_hbm, a1_hbm, a2_hbm, d0_hbm, d1_hbm, d2_hbm = refs[:7]
        h_hbms = refs[7:7 + nt]
        g_hbm = refs[7 + nt]
        sc = refs[8 + nt:]
        idx_v, a0_v, a1_v, a2_v, d0_v, d1_v, d2_v, wtmp_v, out_v = sc[:9]
        rbs = sc[9:9 + ring * nt]
        sems = sc[9 + ring * nt:]
        wid = lax.axis_index("s") * 2 + lax.axis_index("c")
        base = wid * _NPW
        pltpu.sync_copy(idx_hbm.at[pl.ds(base, _NPW)], idx_v)
        pltpu.sync_copy(a0_hbm, a0_v)
        pltpu.sync_copy(a1_hbm, a1_v)
        pltpu.sync_copy(a2_hbm, a2_v)
        pltpu.sync_copy(d0_hbm.at[pl.ds(base, _NPW)], d0_v)
        pltpu.sync_copy(d1_hbm.at[pl.ds(base, _NPW)], d1_v)
        pltpu.sync_copy(d2_hbm.at[pl.ds(base, _NPW)], d2_v)
        for slot in range(ring):
            for t in range(nt):
                pltpu.make_async_copy(h_hbms[t].at[idx_v.at[slot]],
                                      rbs[slot * nt + t],
                                      sems[slot * nt + t]).start()
        li = lax.iota(jnp.int32, 16)
        srcs = [a0_v, a1_v, a2_v]
        dsts = [d0_v, d1_v, d2_v]

        def node_step(n, slot):
            i0 = idx_v[n, pl.ds(0, 16)]
            i1 = idx_v[n, pl.ds(16, 16)]
            nfull = jnp.full((16,), 0, jnp.int32) + n
            for h in range(_H):
                g0 = plsc.load_gather(srcs[h], [i0])
                g1 = plsc.load_gather(srcs[h], [i1])
                dsp = plsc.load_gather(dsts[h], [nfull])
                al0 = g0 + dsp
                al0 = jnp.where(al0 >= 0.0, al0, _NEG * al0)
                al1 = g1 + dsp
                al1 = jnp.where(al1 >= 0.0, al1, _NEG * al1)
                al1 = jnp.where(li == 15, -jnp.inf, al1)
                m = jnp.maximum(jnp.max(al0), jnp.max(al1))
                e0 = jnp.exp(al0 - m)
                e1 = jnp.where(li == 15, 0.0, jnp.exp(al1 - m))
                s = jnp.sum(e0) + jnp.sum(e1) + 1e-16
                wtmp_v[pl.ds(h * 32, 16)] = e0 / s
                wtmp_v[pl.ds(h * 32 + 16, 16)] = e1 / s
            for t in range(nt):
                pltpu.make_async_copy(h_hbms[t].at[idx_v.at[n]],
                                      rbs[slot * nt + t],
                                      sems[slot * nt + t]).wait()
            accs = [jnp.zeros((16,), jnp.float32) for _ in range(nv)]
            for j in range(32):
                ws = [plsc.load_gather(wtmp_v, [jnp.full((16,), h * 32 + j, jnp.int32)])
                      for h in range(_H)]
                c0 = 0
                for t in range(nt):
                    wt = widths[t] // 16
                    for cl in range(min(wt, nv - c0)):
                        c = c0 + cl
                        accs[c] = accs[c] + ws[c // cph] * rbs[slot * nt + t][j, pl.ds(cl * 16, 16)]
                    c0 += wt
            for c in range(nv):
                out_v[n, pl.ds(c * 16, 16)] = accs[c]

            @pl.when(n + ring < _NPW)
            def _():
                for t in range(nt):
                    pltpu.make_async_copy(h_hbms[t].at[idx_v.at[n + ring]],
                                          rbs[slot * nt + t],
                                          sems[slot * nt + t]).start()

        def outer(k, carry):
            for slot in range(ring):
                node_step(ring * k + slot, slot)
            return carry

        lax.fori_loop(0, _NPW // ring, outer, 0)
        pltpu.sync_copy(out_v, g_hbm.at[pl.ds(base, _NPW)])

    return body


def _sc_agg(idx32, asrc, adst, h):
    hc = h.shape[1]
    hp = ((hc + 127) // 128) * 128
    widths, ring = (hp,), 2
    hs = (jnp.pad(h, ((0, 0), (0, hp - hc))) if hp != hc else h,)
    nt = len(widths)
    mesh = plsc.VectorSubcoreMesh(core_axis_name="c", subcore_axis_name="s")
    k = functools.partial(
        pl.kernel,
        mesh=mesh,
        compiler_params=pltpu.CompilerParams(needs_layout_passes=False),
        out_type=jax.ShapeDtypeStruct((_N, hc), jnp.float32),
        scratch_types=[
            pltpu.VMEM((_NPW, 32), jnp.int32),
            pltpu.VMEM((_N,), jnp.float32),
            pltpu.VMEM((_N,), jnp.float32),
            pltpu.VMEM((_N,), jnp.float32),
            pltpu.VMEM((_NPW,), jnp.float32),
            pltpu.VMEM((_NPW,), jnp.float32),
            pltpu.VMEM((_NPW,), jnp.float32),
            pltpu.VMEM((96,), jnp.float32),
            pltpu.VMEM((_NPW, hc), jnp.float32),
        ] + [pltpu.VMEM((32, widths[t]), jnp.float32)
             for _ in range(ring) for t in range(nt)]
          + [pltpu.SemaphoreType.DMA] * (ring * nt),
    )(_sc_agg_body_factory(hc, widths, ring))
    a0, a1, a2 = (asrc[:, i] for i in range(_H))
    d0, d1, d2 = (adst[:, i] for i in range(_H))
    return k(idx32, a0, a1, a2, d0, d1, d2, *hs)


def _proj_body(g_ref, b_ref, wm_ref, bm_ref, o_ref):
    o_ref[...] = jnp.dot(g_ref[...] + b_ref[...], wm_ref[...],
                         preferred_element_type=jnp.float32) + bm_ref[...]


def _gat_sc(xf, idx32, W, att_src, att_dst, bias, out_ch, Wm, bm):
    cin = xf.shape[1]
    hc = _H * out_ch
    blk = 256
    h, asrc, adst = pl.pallas_call(
        _gat_pre_body,
        grid=(_N // blk,),
        in_specs=[
            pl.BlockSpec((blk, cin), lambda i: (i, 0)),
            pl.BlockSpec((cin, hc), lambda i: (0, 0)),
            pl.BlockSpec((1, hc), lambda i: (0, 0)),
            pl.BlockSpec((1, hc), lambda i: (0, 0)),
        ],
        out_specs=[
            pl.BlockSpec((blk, hc), lambda i: (i, 0)),
            pl.BlockSpec((blk, _H), lambda i: (i, 0)),
            pl.BlockSpec((blk, _H), lambda i: (i, 0)),
        ],
        out_shape=[
            jax.ShapeDtypeStruct((_N, hc), jnp.float32),
            jax.ShapeDtypeStruct((_N, _H), jnp.float32),
            jax.ShapeDtypeStruct((_N, _H), jnp.float32),
        ],
    )(xf, W, att_src.reshape(1, hc), att_dst.reshape(1, hc))

    g = _sc_agg(idx32, asrc, adst, h)

    out_ch2 = Wm.shape[1]
    return pl.pallas_call(
        _proj_body,
        grid=(_N // blk,),
        in_specs=[
            pl.BlockSpec((blk, hc), lambda i: (i, 0)),
            pl.BlockSpec((1, hc), lambda i: (0, 0)),
            pl.BlockSpec((hc, out_ch2), lambda i: (0, 0)),
            pl.BlockSpec((1, out_ch2), lambda i: (0, 0)),
        ],
        out_specs=pl.BlockSpec((blk, out_ch2), lambda i: (i, 0)),
        out_shape=jax.ShapeDtypeStruct((_N, out_ch2), jnp.float32),
    )(g, bias.reshape(1, hc), Wm, bm.reshape(1, out_ch2))


def _head_a_body(link_ref, f1_ref, fb1_ref, f2_ref, fb2_ref, o_ref):
    i = pl.program_id(0)
    t = jnp.dot(link_ref[...], f1_ref[...], preferred_element_type=jnp.float32)
    t = jnp.maximum(t + fb1_ref[...], 0.0)
    x5 = jnp.dot(t, f2_ref[...], preferred_element_type=jnp.float32) + fb2_ref[...]
    bmax = jnp.max(x5, axis=0, keepdims=True)

    @pl.when(i == 0)
    def _():
        o_ref[...] = bmax

    @pl.when(i > 0)
    def _():
        o_ref[...] = jnp.maximum(o_ref[...], bmax)


def _head_b_body(link_ref, g_ref, m1a_ref, m1b_ref, mb1_ref, m2_ref, mb2_ref,
                 m3_ref, mb3_ref, m4_ref, mb4_ref, o_ref):
    g2 = jnp.dot(g_ref[...], m1b_ref[...], preferred_element_type=jnp.float32)
    h = jnp.dot(link_ref[...], m1a_ref[...], preferred_element_type=jnp.float32)
    h = jnp.maximum(h + g2 + mb1_ref[...], 0.0)
    h = jnp.dot(h, m2_ref[...], preferred_element_type=jnp.float32)
    h = jnp.maximum(h + mb2_ref[...], 0.0)
    h = jnp.dot(h, m3_ref[...], preferred_element_type=jnp.float32)
    h = jnp.maximum(h + mb3_ref[...], 0.0)
    o = jnp.dot(h, m4_ref[...], preferred_element_type=jnp.float32) + mb4_ref[...]
    m = jnp.max(o, axis=1, keepdims=True)
    lse = jnp.log(jnp.sum(jnp.exp(o - m), axis=1, keepdims=True))
    o_ref[...] = o - m - lse


def _mlp_head(link4, F1, fb1, F2, fb2, M1, mb1, M2, mb2, M3, mb3, M4, mb4):
    blk = 256
    nblk = _N // blk
    cin = link4.shape[1]
    gfeat = pl.pallas_call(
        _head_a_body,
        grid=(nblk,),
        in_specs=[
            pl.BlockSpec((blk, cin), lambda i: (i, 0)),
            pl.BlockSpec((cin, 1024), lambda i: (0, 0)),
            pl.BlockSpec((1, 1024), lambda i: (0, 0)),
            pl.BlockSpec((1024, 1024), lambda i: (0, 0)),
            pl.BlockSpec((1, 1024), lambda i: (0, 0)),
        ],
        out_specs=pl.BlockSpec((1, 1024), lambda i: (0, 0)),
        out_shape=jax.ShapeDtypeStruct((1, 1024), jnp.float32),
    )(link4, F1, fb1.reshape(1, -1), F2, fb2.reshape(1, -1))

    M1a, M1b = M1[:cin], M1[cin:]
    out = pl.pallas_call(
        _head_b_body,
        grid=(nblk,),
        in_specs=[
            pl.BlockSpec((blk, cin), lambda i: (i, 0)),
            pl.BlockSpec((1, 1024), lambda i: (0, 0)),
            pl.BlockSpec((cin, 256), lambda i: (0, 0)),
            pl.BlockSpec((1024, 256), lambda i: (0, 0)),
            pl.BlockSpec((1, 256), lambda i: (0, 0)),
            pl.BlockSpec((256, 256), lambda i: (0, 0)),
            pl.BlockSpec((1, 256), lambda i: (0, 0)),
            pl.BlockSpec((256, 128), lambda i: (0, 0)),
            pl.BlockSpec((1, 128), lambda i: (0, 0)),
            pl.BlockSpec((128, 50), lambda i: (0, 0)),
            pl.BlockSpec((1, 50), lambda i: (0, 0)),
        ],
        out_specs=pl.BlockSpec((blk, 50), lambda i: (i, 0)),
        out_shape=jax.ShapeDtypeStruct((_N, 50), jnp.float32),
    )(link4, gfeat, M1a, M1b, mb1.reshape(1, -1), M2, mb2.reshape(1, -1),
      M3, mb3.reshape(1, -1), M4, mb4.reshape(1, -1))
    return out


def kernel(x, pos, batch, W1, as1, ad1, b1, Wm1, bm1, W2, as2, ad2, b2, Wm2,
           bm2, W3, as3, ad3, b3, Wm3, bm3, W4, as4, ad4, b4, Wm4, bm4, F1,
           fb1, F2, fb2, M1, mb1, M2, mb2, M3, mb3, M4, mb4):
    ar = jnp.arange(_N, dtype=jnp.int32).reshape(_N, 1)
    x0 = jnp.concatenate([x, pos], axis=-1)
    idx32 = jnp.concatenate([_knn_idx(x0, batch), ar, ar], axis=1)
    x1 = _gat_sc(x0, idx32, W1, as1, ad1, b1, 64, Wm1, bm1)
    idx32 = jnp.concatenate([_knn_idx(x1, batch), ar, ar], axis=1)
    link1 = jnp.concatenate([x0, x1], axis=1)
    x2 = _gat_sc(link1, idx32, W2, as2, ad2, b2, 64, Wm2, bm2)
    idx32 = jnp.concatenate([_knn_idx(x2, batch), ar, ar], axis=1)
    link2 = jnp.concatenate([x0, x1, x2], axis=1)
    x3 = _gat_sc(link2, idx32, W3, as3, ad3, b3, 64, Wm3, bm3)
    link3 = jnp.concatenate([x0, x1, x2, x3], axis=1)
    x4 = _gat_sc(link3, idx32, W4, as4, ad4, b4, 128, Wm4, bm4)
    link4 = jnp.concatenate([x0, x1, x2, x3, x4], axis=-1)
    return _mlp_head(link4, F1, fb1, F2, fb2, M1, mb1, M2, mb2, M3, mb3, M4, mb4)
